# baseline (speedup 1.0000x reference)
import functools
import jax
import jax.numpy as jnp
from jax import lax
from jax.experimental import pallas as pl
from jax.experimental.pallas import tpu as pltpu
from jax.experimental.pallas import tpu_sc as plsc

N = 50000
E = 800000
NG = 64
H = 64
NP = 51200
EP = 819200
NWORK = 32
EPW = EP // NWORK
CH = 128
STR = NP // 16
BN = 2000
GRID = N // BN
NEG = -1e30

_MESH = plsc.VectorSubcoreMesh(
    core_axis_name="c", subcore_axis_name="s", num_cores=2, num_subcores=16
)
_SC_PARAMS = pltpu.CompilerParams(
    needs_layout_passes=False, use_tc_tiling_on_sc=False
)


def _sc1_body(q_hbm, k_hbm, src_hbm, dst_hbm, alpha_out, cpart_out,
              sidx, didx, qrows, krows, abuf, cbuf, cacc, ctmp, cshared, sem):
    c = lax.axis_index("c")
    s = lax.axis_index("s")
    base = (c * 16 + s) * EPW
    row16 = lax.iota(jnp.int32, 16)
    neg16 = jnp.full((16,), NEG, jnp.float32)

    def init_c(i, carry):
        cbuf[pl.ds(i * 16, 16)] = neg16
        return carry
    lax.fori_loop(0, NP // 16, init_c, 0)

    def chunk(g, carry):
        off = base + g * CH
        pltpu.sync_copy(src_hbm.at[pl.ds(off, CH)], sidx)
        pltpu.sync_copy(dst_hbm.at[pl.ds(off, CH)], didx)
        cp1 = pltpu.async_copy(q_hbm.at[didx], qrows, sem)
        cp2 = pltpu.async_copy(k_hbm.at[sidx], krows, sem)
        cp1.wait()
        cp2.wait()
        for i in range(CH // 16):
            ridx = row16 + (i * 16)
            acc = jnp.zeros((16,), jnp.float32)
            for f in range(H):
                fv = jnp.full((16,), f, jnp.int32)
                acc = acc + (plsc.load_gather(qrows, [ridx, fv]) *
                             plsc.load_gather(krows, [ridx, fv]))
            a16 = acc * 0.125
            eid = off + i * 16 + row16
            a16 = jnp.where(eid < E, a16, NEG)
            abuf[pl.ds(i * 16, 16)] = a16
            d16 = didx[pl.ds(i * 16, 16)]
            cur = plsc.load_gather(cbuf, [d16])
            plsc.store_scatter(cbuf, [d16], jnp.maximum(cur, a16))
        pltpu.sync_copy(abuf, alpha_out.at[pl.ds(off, CH)])
        return carry
    lax.fori_loop(0, EPW // CH, chunk, 0)

    pltpu.sync_copy(cbuf, cshared.at[c * 16 + s])
    plsc.subcore_barrier()
    sb = s * STR
    pltpu.sync_copy(cshared.at[c * 16, pl.ds(sb, STR)], cacc)
    for j in range(1, 16):
        pltpu.sync_copy(cshared.at[c * 16 + j, pl.ds(sb, STR)], ctmp)

        def mx(i, carry):
            o = i * 16
            cacc[pl.ds(o, 16)] = jnp.maximum(cacc[pl.ds(o, 16)],
                                             ctmp[pl.ds(o, 16)])
            return carry
        lax.fori_loop(0, STR // 16, mx, 0)

    @pl.when(c == 0)
    def _():
        pltpu.sync_copy(cacc, cpart_out.at[0, pl.ds(sb, STR)])

    @pl.when(c == 1)
    def _():
        pltpu.sync_copy(cacc, cpart_out.at[1, pl.ds(sb, STR)])


_sc1 = functools.partial(
    pl.kernel,
    out_type=[jax.ShapeDtypeStruct((EP,), jnp.float32),
              jax.ShapeDtypeStruct((2, NP), jnp.float32)],
    mesh=_MESH,
    compiler_params=_SC_PARAMS,
    scratch_types=[
        pltpu.VMEM((CH,), jnp.int32),
        pltpu.VMEM((CH,), jnp.int32),
        pltpu.VMEM((CH, H), jnp.float32),
        pltpu.VMEM((CH, H), jnp.float32),
        pltpu.VMEM((CH,), jnp.float32),
        pltpu.VMEM((NP,), jnp.float32),
        pltpu.VMEM((STR,), jnp.float32),
        pltpu.VMEM((STR,), jnp.float32),
        pltpu.MemorySpace.HBM((32, NP), jnp.float32),
        pltpu.SemaphoreType.DMA,
    ],
)(_sc1_body)


def _sc2_body(alpha_hbm, dst_hbm, cpart_hbm, e_out, spart_out,
              didx, abuf, ebuf, cbuf, sbuf, ctmp, sacc, sshared, sem):
    c = lax.axis_index("c")
    s = lax.axis_index("s")
    base = (c * 16 + s) * EPW
    row16 = lax.iota(jnp.int32, 16)
    zero16 = jnp.zeros((16,), jnp.float32)

    pltpu.sync_copy(cpart_hbm.at[0], cbuf)
    for st in range(16):
        pltpu.sync_copy(cpart_hbm.at[1, pl.ds(st * STR, STR)], ctmp)

        def mx(i, carry):
            o = i * 16
            cbuf[pl.ds(st * STR + o, 16)] = jnp.maximum(
                cbuf[pl.ds(st * STR + o, 16)], ctmp[pl.ds(o, 16)])
            return carry
        lax.fori_loop(0, STR // 16, mx, 0)

    def init_s(i, carry):
        sbuf[pl.ds(i * 16, 16)] = zero16
        return carry
    lax.fori_loop(0, NP // 16, init_s, 0)

    def chunk(g, carry):
        off = base + g * CH
        pltpu.sync_copy(dst_hbm.at[pl.ds(off, CH)], didx)
        pltpu.sync_copy(alpha_hbm.at[pl.ds(off, CH)], abuf)
        for i in range(CH // 16):
            o = i * 16
            a16 = abuf[pl.ds(o, 16)]
            d16 = didx[pl.ds(o, 16)]
            c16 = plsc.load_gather(cbuf, [d16])
            e16 = jnp.exp(a16 - c16)
            eid = off + o + row16
            e16 = jnp.where(eid < E, e16, 0.0)
            ebuf[pl.ds(o, 16)] = e16
            plsc.addupdate_scatter(sbuf, [d16], e16)
        pltpu.sync_copy(ebuf, e_out.at[pl.ds(off, CH)])
        return carry
    lax.fori_loop(0, EPW // CH, chunk, 0)

    pltpu.sync_copy(sbuf, sshared.at[c * 16 + s])
    plsc.subcore_barrier()
    sb = s * STR
    pltpu.sync_copy(sshared.at[c * 16, pl.ds(sb, STR)], sacc)
    for j in range(1, 16):
        pltpu.sync_copy(sshared.at[c * 16 + j, pl.ds(sb, STR)], ctmp)

        def ad(i, carry):
            o = i * 16
            sacc[pl.ds(o, 16)] = sacc[pl.ds(o, 16)] + ctmp[pl.ds(o, 16)]
            return carry
        lax.fori_loop(0, STR // 16, ad, 0)

    @pl.when(c == 0)
    def _():
        pltpu.sync_copy(sacc, spart_out.at[0, pl.ds(sb, STR)])

    @pl.when(c == 1)
    def _():
        pltpu.sync_copy(sacc, spart_out.at[1, pl.ds(sb, STR)])


_sc2 = functools.partial(
    pl.kernel,
    out_type=[jax.ShapeDtypeStruct((EP,), jnp.float32),
              jax.ShapeDtypeStruct((2, NP), jnp.float32)],
    mesh=_MESH,
    compiler_params=_SC_PARAMS,
    scratch_types=[
        pltpu.VMEM((CH,), jnp.int32),
        pltpu.VMEM((CH,), jnp.float32),
        pltpu.VMEM((CH,), jnp.float32),
        pltpu.VMEM((NP,), jnp.float32),
        pltpu.VMEM((NP,), jnp.float32),
        pltpu.VMEM((STR,), jnp.float32),
        pltpu.VMEM((STR,), jnp.float32),
        pltpu.MemorySpace.HBM((32, NP), jnp.float32),
        pltpu.SemaphoreType.DMA,
    ],
)(_sc2_body)


def _sc3_body(e_hbm, src_hbm, dst_hbm, vstk_hbm, agg_out,
              sidx, didx, ebuf, vrows, aggsh, sem):
    c = lax.axis_index("c")
    s = lax.axis_index("s")
    ept = EP // 16
    base = s * ept
    row16 = lax.iota(jnp.int32, 16)
    zero16 = jnp.zeros((16,), jnp.float32)
    coff = jnp.zeros((16,), jnp.int32) + c * N

    for i in range(CH):
        for j in range(2):
            vrows[i, pl.ds(j * 16, 16)] = zero16
    for t in range(STR // CH):
        pltpu.sync_copy(vrows, aggsh.at[pl.ds(s * STR + t * CH, CH)])
    plsc.subcore_barrier()

    def chunk(g, carry):
        off = base + g * CH
        pltpu.sync_copy(src_hbm.at[pl.ds(off, CH)], sidx)
        pltpu.sync_copy(dst_hbm.at[pl.ds(off, CH)], didx)
        pltpu.sync_copy(e_hbm.at[pl.ds(off, CH)], ebuf)
        for i in range(CH // 16):
            o = i * 16
            sidx[pl.ds(o, 16)] = sidx[pl.ds(o, 16)] + coff
        pltpu.async_copy(vstk_hbm.at[sidx], vrows, sem).wait()
        for i in range(CH // 16):
            e16 = ebuf[pl.ds(i * 16, 16)]
            ridx = row16 + (i * 16)
            for col in range(H // 2):
                cv = jnp.full((16,), col, jnp.int32)
                v16 = plsc.load_gather(vrows, [ridx, cv])
                plsc.store_scatter(vrows, [ridx, cv], v16 * e16)
        pltpu.sync_copy(vrows, aggsh.at[didx], add=True)
        return carry
    lax.fori_loop(0, ept // CH, chunk, 0)
    plsc.subcore_barrier()

    sl = pl.ds(s * STR, STR)
    pltpu.sync_copy(aggsh.at[sl], agg_out.at[c, sl])


_sc3 = functools.partial(
    pl.kernel,
    out_type=jax.ShapeDtypeStruct((2, NP, H // 2), jnp.float32),
    mesh=_MESH,
    compiler_params=_SC_PARAMS,
    scratch_types=[
        pltpu.VMEM((CH,), jnp.int32),
        pltpu.VMEM((CH,), jnp.int32),
        pltpu.VMEM((CH,), jnp.float32),
        pltpu.VMEM((CH, H // 2), jnp.float32),
        pltpu.VMEM_SHARED((NP, H // 2), jnp.float32),
        pltpu.SemaphoreType.DMA,
    ],
)(_sc3_body)


def _tca_body(gt, ar, di, gn, emb, W_in, b_in, Wq, bq, Wk, bk, Wv, bv,
              Ws, bs, q_o, k_o, va_o, vb_o, so_o, xp_o):
    iot = lax.broadcasted_iota(jnp.int32, (1, 31), 1).astype(jnp.float32)
    oh = (gt[...] == iot).astype(jnp.float32)
    a = ar[...]
    d = di[...]
    g = gn[...]

    def lin(W, b):
        ew = jnp.dot(emb[...], W[0:16, :], preferred_element_type=jnp.float32)
        t = jnp.dot(oh, ew, preferred_element_type=jnp.float32)
        return (t + a * W[16:17, :] + d * W[17:18, :] + g * W[18:19, :]
                + b[...])

    q_o[...] = lin(Wq, bq)
    k_o[...] = lin(Wk, bk)
    v = lin(Wv, bv)
    va_o[...] = v[:, :H // 2]
    vb_o[...] = v[:, H // 2:]
    so_o[...] = lin(Ws, bs)
    xp_o[...] = lin(W_in, b_in)


def _tca(gtf, arf, dif, gnf, emb, W_in, b_in, Wq, bq, Wk, bk, Wv, bv, Ws, bs):
    col = pl.BlockSpec((BN, 1), lambda i: (i, 0))
    full = lambda sh: pl.BlockSpec(sh, lambda i: tuple(0 for _ in sh))
    wspec = full((19, H))
    bspec = full((1, H))
    out64 = pl.BlockSpec((BN, H), lambda i: (i, 0))
    out32 = pl.BlockSpec((BN, H // 2), lambda i: (i, 0))
    return pl.pallas_call(
        _tca_body,
        grid=(GRID,),
        in_specs=[col, col, col, col, full((31, 16)), wspec, bspec,
                  wspec, bspec, wspec, bspec, wspec, bspec, wspec, bspec],
        out_specs=[out64, out64, out32, out32, out64, out64],
        out_shape=[jax.ShapeDtypeStruct((N, H), jnp.float32),
                   jax.ShapeDtypeStruct((N, H), jnp.float32),
                   jax.ShapeDtypeStruct((N, H // 2), jnp.float32),
                   jax.ShapeDtypeStruct((N, H // 2), jnp.float32),
                   jax.ShapeDtypeStruct((N, H), jnp.float32),
                   jax.ShapeDtypeStruct((N, H), jnp.float32)],
    )(gtf, arf, dif, gnf, emb, W_in, b_in, Wq, bq, Wk, bk, Wv, bv, Ws, bs)


def _tcb_body(agga, aggb, sp, so, res, lg, lb, Wq, bq, Wk, bk, Wv, bv,
              Ws, bs, x1_o, q_o, k_o, va_o, vb_o, so_o):
    spv = sp[...]
    ssum = spv[:, 0:1] + spv[:, 1:2]
    inv = jnp.where(ssum > 0.0, 1.0 / ssum, 0.0)
    h = jnp.concatenate([agga[...], aggb[...]], axis=1) * inv + so[...]
    mu = jnp.mean(h, axis=1, keepdims=True)
    var = jnp.mean((h - mu) ** 2, axis=1, keepdims=True)
    h = (h - mu) * lax.rsqrt(var + 1e-5) * lg[...] + lb[...]
    x1 = h * jax.nn.sigmoid(h) + res[...]
    x1_o[...] = x1

    def lin(W, b):
        return jnp.dot(x1, W[...], preferred_element_type=jnp.float32) + b[...]

    q_o[...] = lin(Wq, bq)
    k_o[...] = lin(Wk, bk)
    v = lin(Wv, bv)
    va_o[...] = v[:, :H // 2]
    vb_o[...] = v[:, H // 2:]
    so_o[...] = lin(Ws, bs)


def _tcb(agga, aggb, sp, so, res, lg, lb, Wq, bq, Wk, bk, Wv, bv, Ws, bs):
    blk64 = pl.BlockSpec((BN, H), lambda i: (i, 0))
    blk32 = pl.BlockSpec((BN, H // 2), lambda i: (i, 0))
    blk2 = pl.BlockSpec((BN, 2), lambda i: (i, 0))
    full = lambda sh: pl.BlockSpec(sh, lambda i: tuple(0 for _ in sh))
    wspec = full((H, H))
    bspec = full((1, H))
    return pl.pallas_call(
        _tcb_body,
        grid=(GRID,),
        in_specs=[blk32, blk32, blk2, blk64, blk64, bspec, bspec,
                  wspec, bspec, wspec, bspec, wspec, bspec, wspec, bspec],
        out_specs=[blk64, blk64, blk64, blk32, blk32, blk64],
        out_shape=[jax.ShapeDtypeStruct((N, H), jnp.float32)] * 3
        + [jax.ShapeDtypeStruct((N, H // 2), jnp.float32)] * 2
        + [jax.ShapeDtypeStruct((N, H), jnp.float32)],
    )(agga, aggb, sp, so, res, lg, lb, Wq, bq, Wk, bk, Wv, bv, Ws, bs)


def _tcc_body(agga, aggb, sp, so, res, lg, lb, bf, bb, pb, W_bb, b_bb,
              W_th, b_th, W_rt, b_rt, W_a1, b_a1, W_a2, b_a2,
              th_o, rt_o, aux_o, sum_r, sq_r, mx_r, cnt_r):
    pid = pl.program_id(0)
    spv = sp[...]
    ssum = spv[:, 0:1] + spv[:, 1:2]
    inv = jnp.where(ssum > 0.0, 1.0 / ssum, 0.0)
    h = jnp.concatenate([agga[...], aggb[...]], axis=1) * inv + so[...]
    mu = jnp.mean(h, axis=1, keepdims=True)
    var = jnp.mean((h - mu) ** 2, axis=1, keepdims=True)
    h = (h - mu) * lax.rsqrt(var + 1e-5) * lg[...] + lb[...]
    x2 = h * jax.nn.sigmoid(h) + res[...]

    @pl.when(pid == 0)
    def _():
        sum_r[...] = jnp.zeros_like(sum_r)
        sq_r[...] = jnp.zeros_like(sq_r)
        mx_r[...] = jnp.full_like(mx_r, NEG)
        cnt_r[...] = jnp.zeros_like(cnt_r)

    b = bf[...]
    iot = lax.broadcasted_iota(jnp.int32, (1, NG), 1).astype(jnp.float32)
    oh = (b == iot).astype(jnp.float32)
    dn = (((0,), (0,)), ((), ()))
    sum_r[...] += lax.dot_general(oh, x2, dn,
                                  preferred_element_type=jnp.float32)
    sq_r[...] += lax.dot_general(oh, x2 * x2, dn,
                                 preferred_element_type=jnp.float32)
    cnt_r[...] += jnp.sum(oh, axis=0, keepdims=True)

    def upd(g, carry):
        ohg = (b == g.astype(jnp.float32))
        m = jnp.max(jnp.where(ohg, x2, NEG), axis=0, keepdims=True)
        mx_r[pl.ds(g, 1), :] = jnp.maximum(mx_r[pl.ds(g, 1), :], m)
        return carry
    lax.fori_loop(0, NG, upd, 0)

    @pl.when(pid == GRID - 1)
    def _():
        cnt = jnp.transpose(cnt_r[...])
        cnt1 = jnp.maximum(cnt, 1.0)
        mean = sum_r[...] / cnt1
        msq = sq_r[...] / cnt1
        std = jnp.sqrt(jnp.clip(msq - mean * mean, 1e-6, None))
        mx = jnp.where(cnt > 0.0, mx_r[...], 0.0)
        gnn = jnp.concatenate([mean, mx, std], axis=1)
        comb = jnp.concatenate([gnn, bb[...], pb[...]], axis=1)
        f = jnp.dot(comb, W_bb[...], preferred_element_type=jnp.float32)
        f = f + b_bb[...]
        f = f * jax.nn.sigmoid(f)
        th_o[...] = jnp.dot(f, W_th[...],
                            preferred_element_type=jnp.float32) + b_th[...]
        rt_o[...] = jnp.dot(f, W_rt[...],
                            preferred_element_type=jnp.float32) + b_rt[...]
        a1 = jnp.dot(gnn, W_a1[...],
                     preferred_element_type=jnp.float32) + b_a1[...]
        a1 = a1 * jax.nn.sigmoid(a1)
        aux_o[...] = jnp.dot(a1, W_a2[...],
                             preferred_element_type=jnp.float32) + b_a2[...]


def _tcc(agga, aggb, sp, so, res, lg, lb, bf, bb, pb, W_bb, b_bb,
         W_th, b_th, W_rt, b_rt, W_a1, b_a1, W_a2, b_a2):
    blk64 = pl.BlockSpec((BN, H), lambda i: (i, 0))
    blk32 = pl.BlockSpec((BN, H // 2), lambda i: (i, 0))
    blk2 = pl.BlockSpec((BN, 2), lambda i: (i, 0))
    col = pl.BlockSpec((BN, 1), lambda i: (i, 0))
    full = lambda sh: pl.BlockSpec(sh, lambda i: tuple(0 for _ in sh))
    return pl.pallas_call(
        _tcc_body,
        grid=(GRID,),
        in_specs=[blk32, blk32, blk2, blk64, blk64, full((1, H)),
                  full((1, H)), col, full((NG, 1)), full((NG, 1)),
                  full((3 * H + 2, H)), full((1, H)),
                  full((H, 10)), full((1, 10)),
                  full((H, 1)), full((1, 1)),
                  full((3 * H, H)), full((1, H)),
                  full((H, 32)), full((1, 32))],
        out_specs=[full((NG, 10)), full((NG, 1)), full((NG, 32))],
        out_shape=[jax.ShapeDtypeStruct((NG, 10), jnp.float32),
                   jax.ShapeDtypeStruct((NG, 1), jnp.float32),
                   jax.ShapeDtypeStruct((NG, 32), jnp.float32)],
        scratch_shapes=[pltpu.VMEM((NG, H), jnp.float32),
                        pltpu.VMEM((NG, H), jnp.float32),
                        pltpu.VMEM((NG, H), jnp.float32),
                        pltpu.VMEM((1, NG), jnp.float32)],
    )(agga, aggb, sp, so, res, lg, lb, bf, bb, pb, W_bb, b_bb,
      W_th, b_th, W_rt, b_rt, W_a1, b_a1, W_a2, b_a2)


def kernel(gate_type_idx, gate_arity, is_directional, gate_index_norm,
           edge_index, batch, backend_bit, precision_bit, emb, W_in, b_in,
           Wq0, bq0, Wk0, bk0, Wv0, bv0, Ws0, bs0, ln0_g, ln0_b,
           Wq1, bq1, Wk1, bk1, Wv1, bv1, Ws1, bs1, ln1_g, ln1_b,
           W_bb, b_bb, W_th, b_th, W_rt, b_rt, W_a1, b_a1, W_a2, b_a2):
    f32 = jnp.float32
    gtf = gate_type_idx.astype(f32).reshape(N, 1)
    arf = gate_arity.astype(f32).reshape(N, 1)
    dif = is_directional.astype(f32).reshape(N, 1)
    gnf = gate_index_norm.reshape(N, 1)
    bf = batch.astype(f32).reshape(N, 1)
    bb = backend_bit.reshape(NG, 1)
    pb = precision_bit.reshape(NG, 1)
    r1 = lambda v: v.reshape(1, -1)

    zpad = jnp.zeros((EP - E,), jnp.int32)
    src = jnp.concatenate([edge_index[0].astype(jnp.int32), zpad])
    dst = jnp.concatenate([edge_index[1].astype(jnp.int32), zpad])

    q0, k0, va0, vb0, so0, xproj = _tca(
        gtf, arf, dif, gnf, emb, W_in, r1(b_in), Wq0, r1(bq0), Wk0, r1(bk0),
        Wv0, r1(bv0), Ws0, r1(bs0))

    def edge_phase(q, k, va, vb):
        alpha, cpart = _sc1(q, k, src, dst)
        e, spart = _sc2(alpha, dst, cpart)
        vstk = jnp.concatenate([va, vb], axis=0)
        agg = _sc3(e, src, dst, vstk)
        sp = jnp.transpose(spart)[:N]
        return agg[0, :N], agg[1, :N], sp

    agga0, aggb0, sp0 = edge_phase(q0, k0, va0, vb0)

    x1, q1, k1, va1, vb1, so1 = _tcb(
        agga0, aggb0, sp0, so0, xproj, r1(ln0_g), r1(ln0_b),
        Wq1, r1(bq1), Wk1, r1(bk1), Wv1, r1(bv1), Ws1, r1(bs1))

    agga1, aggb1, sp1 = edge_phase(q1, k1, va1, vb1)

    th, rt, aux = _tcc(
        agga1, aggb1, sp1, so1, x1, r1(ln1_g), r1(ln1_b), bf, bb, pb,
        W_bb, r1(b_bb), W_th, r1(b_th), W_rt, r1(b_rt),
        W_a1, r1(b_a1), W_a2, r1(b_a2))
    return (th, rt[:, 0], aux)


# speedup vs baseline: 1.0122x; 1.0122x over previous
import functools
import jax
import jax.numpy as jnp
from jax import lax
from jax.experimental import pallas as pl
from jax.experimental.pallas import tpu as pltpu
from jax.experimental.pallas import tpu_sc as plsc

N = 50000
E = 800000
NG = 64
H = 64
NP = 51200
EP = 819200
NWORK = 32
EPW = EP // NWORK
CH = 128
STR = NP // 16
BN = 2000
GRID = N // BN
NEG = -1e30

_MESH = plsc.VectorSubcoreMesh(
    core_axis_name="c", subcore_axis_name="s", num_cores=2, num_subcores=16
)
_SC_PARAMS = pltpu.CompilerParams(
    needs_layout_passes=False, use_tc_tiling_on_sc=False
)


def _sc1_body(q_hbm, k_hbm, src_hbm, dst_hbm, alpha_out, cpart_out,
              sidx, didx, didxf, qrows, krows, abuf, cbuf, cacc, ctmp, cshared,
              sem):
    c = lax.axis_index("c")
    s = lax.axis_index("s")
    base = (c * 16 + s) * EPW
    row16 = lax.iota(jnp.int32, 16)
    neg16 = jnp.full((16,), NEG, jnp.float32)

    def init_c(i, carry):
        cbuf[pl.ds(i * 16, 16)] = neg16
        return carry
    lax.fori_loop(0, NP // 16, init_c, 0)

    def chunk(g, carry):
        off = base + g * (4 * CH)
        pltpu.sync_copy(dst_hbm.at[pl.ds(off, 4 * CH)], didxf)
        for b in range(4):
            pltpu.sync_copy(src_hbm.at[pl.ds(off + b * CH, CH)], sidx.at[b])
            pltpu.sync_copy(dst_hbm.at[pl.ds(off + b * CH, CH)], didx.at[b])
        cps = []
        for b in range(4):
            cps.append(pltpu.async_copy(
                q_hbm.at[didx.at[b]], qrows.at[pl.ds(b * CH, CH)], sem))
            cps.append(pltpu.async_copy(
                k_hbm.at[sidx.at[b]], krows.at[pl.ds(b * CH, CH)], sem))
        for cp in cps:
            cp.wait()

        def grp(i, carry2):
            ridx = row16 + i * 16
            acc = jnp.zeros((16,), jnp.float32)
            for f in range(H):
                fv = jnp.full((16,), f, jnp.int32)
                acc = acc + (plsc.load_gather(qrows, [ridx, fv]) *
                             plsc.load_gather(krows, [ridx, fv]))
            a16 = acc * 0.125
            eid = off + i * 16 + row16
            a16 = jnp.where(eid < E, a16, NEG)
            abuf[pl.ds(i * 16, 16)] = a16
            d16 = didxf[pl.ds(i * 16, 16)]
            cur = plsc.load_gather(cbuf, [d16])
            plsc.store_scatter(cbuf, [d16], jnp.maximum(cur, a16))
            return carry2
        lax.fori_loop(0, 4 * CH // 16, grp, 0)
        pltpu.sync_copy(abuf, alpha_out.at[pl.ds(off, 4 * CH)])
        return carry
    lax.fori_loop(0, EPW // (4 * CH), chunk, 0)

    pltpu.sync_copy(cbuf, cshared.at[c * 16 + s])
    plsc.subcore_barrier()
    sb = s * STR
    pltpu.sync_copy(cshared.at[c * 16, pl.ds(sb, STR)], cacc)
    for j in range(1, 16):
        pltpu.sync_copy(cshared.at[c * 16 + j, pl.ds(sb, STR)], ctmp)

        def mx(i, carry):
            o = i * 16
            cacc[pl.ds(o, 16)] = jnp.maximum(cacc[pl.ds(o, 16)],
                                             ctmp[pl.ds(o, 16)])
            return carry
        lax.fori_loop(0, STR // 16, mx, 0)

    @pl.when(c == 0)
    def _():
        pltpu.sync_copy(cacc, cpart_out.at[0, pl.ds(sb, STR)])

    @pl.when(c == 1)
    def _():
        pltpu.sync_copy(cacc, cpart_out.at[1, pl.ds(sb, STR)])


_sc1 = functools.partial(
    pl.kernel,
    out_type=[jax.ShapeDtypeStruct((EP,), jnp.float32),
              jax.ShapeDtypeStruct((2, NP), jnp.float32)],
    mesh=_MESH,
    compiler_params=_SC_PARAMS,
    scratch_types=[
        pltpu.VMEM((4, CH), jnp.int32),
        pltpu.VMEM((4, CH), jnp.int32),
        pltpu.VMEM((4 * CH,), jnp.int32),
        pltpu.VMEM((4 * CH, H), jnp.float32),
        pltpu.VMEM((4 * CH, H), jnp.float32),
        pltpu.VMEM((4 * CH,), jnp.float32),
        pltpu.VMEM((NP,), jnp.float32),
        pltpu.VMEM((STR,), jnp.float32),
        pltpu.VMEM((STR,), jnp.float32),
        pltpu.MemorySpace.HBM((32, NP), jnp.float32),
        pltpu.SemaphoreType.DMA,
    ],
)(_sc1_body)


def _sc2_body(alpha_hbm, dst_hbm, cpart_hbm, e_out, spart_out,
              didx, abuf, ebuf, cbuf, sbuf, ctmp, sacc, sshared, sem):
    c = lax.axis_index("c")
    s = lax.axis_index("s")
    base = (c * 16 + s) * EPW
    row16 = lax.iota(jnp.int32, 16)
    zero16 = jnp.zeros((16,), jnp.float32)

    pltpu.sync_copy(cpart_hbm.at[0], cbuf)
    for st in range(16):
        pltpu.sync_copy(cpart_hbm.at[1, pl.ds(st * STR, STR)], ctmp)

        def mx(i, carry):
            o = i * 16
            cbuf[pl.ds(st * STR + o, 16)] = jnp.maximum(
                cbuf[pl.ds(st * STR + o, 16)], ctmp[pl.ds(o, 16)])
            return carry
        lax.fori_loop(0, STR // 16, mx, 0)

    def init_s(i, carry):
        sbuf[pl.ds(i * 16, 16)] = zero16
        return carry
    lax.fori_loop(0, NP // 16, init_s, 0)

    def chunk(g, carry):
        off = base + g * CH
        pltpu.sync_copy(dst_hbm.at[pl.ds(off, CH)], didx)
        pltpu.sync_copy(alpha_hbm.at[pl.ds(off, CH)], abuf)
        for i in range(CH // 16):
            o = i * 16
            a16 = abuf[pl.ds(o, 16)]
            d16 = didx[pl.ds(o, 16)]
            c16 = plsc.load_gather(cbuf, [d16])
            e16 = jnp.exp(a16 - c16)
            eid = off + o + row16
            e16 = jnp.where(eid < E, e16, 0.0)
            ebuf[pl.ds(o, 16)] = e16
            plsc.addupdate_scatter(sbuf, [d16], e16)
        pltpu.sync_copy(ebuf, e_out.at[pl.ds(off, CH)])
        return carry
    lax.fori_loop(0, EPW // CH, chunk, 0)

    pltpu.sync_copy(sbuf, sshared.at[c * 16 + s])
    plsc.subcore_barrier()
    sb = s * STR
    pltpu.sync_copy(sshared.at[c * 16, pl.ds(sb, STR)], sacc)
    for j in range(1, 16):
        pltpu.sync_copy(sshared.at[c * 16 + j, pl.ds(sb, STR)], ctmp)

        def ad(i, carry):
            o = i * 16
            sacc[pl.ds(o, 16)] = sacc[pl.ds(o, 16)] + ctmp[pl.ds(o, 16)]
            return carry
        lax.fori_loop(0, STR // 16, ad, 0)

    @pl.when(c == 0)
    def _():
        pltpu.sync_copy(sacc, spart_out.at[0, pl.ds(sb, STR)])

    @pl.when(c == 1)
    def _():
        pltpu.sync_copy(sacc, spart_out.at[1, pl.ds(sb, STR)])


_sc2 = functools.partial(
    pl.kernel,
    out_type=[jax.ShapeDtypeStruct((EP,), jnp.float32),
              jax.ShapeDtypeStruct((2, NP), jnp.float32)],
    mesh=_MESH,
    compiler_params=_SC_PARAMS,
    scratch_types=[
        pltpu.VMEM((CH,), jnp.int32),
        pltpu.VMEM((CH,), jnp.float32),
        pltpu.VMEM((CH,), jnp.float32),
        pltpu.VMEM((NP,), jnp.float32),
        pltpu.VMEM((NP,), jnp.float32),
        pltpu.VMEM((STR,), jnp.float32),
        pltpu.VMEM((STR,), jnp.float32),
        pltpu.MemorySpace.HBM((32, NP), jnp.float32),
        pltpu.SemaphoreType.DMA,
    ],
)(_sc2_body)


def _sc3_body(e_hbm, src_hbm, dst_hbm, vstk_hbm, agg_out,
              sidx, didx, ebuf, vrows, aggsh, sem):
    c = lax.axis_index("c")
    s = lax.axis_index("s")
    ept = EP // 16
    base = s * ept
    row16 = lax.iota(jnp.int32, 16)
    zero16 = jnp.zeros((16,), jnp.float32)
    coff = jnp.zeros((16,), jnp.int32) + c * N

    for i in range(CH):
        for j in range(2):
            vrows[i, pl.ds(j * 16, 16)] = zero16
    for t in range(STR // CH):
        pltpu.sync_copy(vrows, aggsh.at[pl.ds(s * STR + t * CH, CH)])
    plsc.subcore_barrier()

    def chunk(g, carry):
        off = base + g * CH
        pltpu.sync_copy(src_hbm.at[pl.ds(off, CH)], sidx)
        pltpu.sync_copy(dst_hbm.at[pl.ds(off, CH)], didx)
        pltpu.sync_copy(e_hbm.at[pl.ds(off, CH)], ebuf)
        for i in range(CH // 16):
            o = i * 16
            sidx[pl.ds(o, 16)] = sidx[pl.ds(o, 16)] + coff
        pltpu.async_copy(vstk_hbm.at[sidx], vrows, sem).wait()
        for i in range(CH // 16):
            e16 = ebuf[pl.ds(i * 16, 16)]
            ridx = row16 + (i * 16)
            for col in range(H // 2):
                cv = jnp.full((16,), col, jnp.int32)
                v16 = plsc.load_gather(vrows, [ridx, cv])
                plsc.store_scatter(vrows, [ridx, cv], v16 * e16)
        pltpu.sync_copy(vrows, aggsh.at[didx], add=True)
        return carry
    lax.fori_loop(0, ept // CH, chunk, 0)
    plsc.subcore_barrier()

    sl = pl.ds(s * STR, STR)
    pltpu.sync_copy(aggsh.at[sl], agg_out.at[c, sl])


_sc3 = functools.partial(
    pl.kernel,
    out_type=jax.ShapeDtypeStruct((2, NP, H // 2), jnp.float32),
    mesh=_MESH,
    compiler_params=_SC_PARAMS,
    scratch_types=[
        pltpu.VMEM((CH,), jnp.int32),
        pltpu.VMEM((CH,), jnp.int32),
        pltpu.VMEM((CH,), jnp.float32),
        pltpu.VMEM((CH, H // 2), jnp.float32),
        pltpu.VMEM_SHARED((NP, H // 2), jnp.float32),
        pltpu.SemaphoreType.DMA,
    ],
)(_sc3_body)


def _tca_body(gt, ar, di, gn, emb, W_in, b_in, Wq, bq, Wk, bk, Wv, bv,
              Ws, bs, q_o, k_o, va_o, vb_o, so_o, xp_o):
    iot = lax.broadcasted_iota(jnp.int32, (1, 31), 1).astype(jnp.float32)
    oh = (gt[...] == iot).astype(jnp.float32)
    a = ar[...]
    d = di[...]
    g = gn[...]

    def lin(W, b):
        ew = jnp.dot(emb[...], W[0:16, :], preferred_element_type=jnp.float32)
        t = jnp.dot(oh, ew, preferred_element_type=jnp.float32)
        return (t + a * W[16:17, :] + d * W[17:18, :] + g * W[18:19, :]
                + b[...])

    q_o[...] = lin(Wq, bq)
    k_o[...] = lin(Wk, bk)
    v = lin(Wv, bv)
    va_o[...] = v[:, :H // 2]
    vb_o[...] = v[:, H // 2:]
    so_o[...] = lin(Ws, bs)
    xp_o[...] = lin(W_in, b_in)


def _tca(gtf, arf, dif, gnf, emb, W_in, b_in, Wq, bq, Wk, bk, Wv, bv, Ws, bs):
    col = pl.BlockSpec((BN, 1), lambda i: (i, 0))
    full = lambda sh: pl.BlockSpec(sh, lambda i: tuple(0 for _ in sh))
    wspec = full((19, H))
    bspec = full((1, H))
    out64 = pl.BlockSpec((BN, H), lambda i: (i, 0))
    out32 = pl.BlockSpec((BN, H // 2), lambda i: (i, 0))
    return pl.pallas_call(
        _tca_body,
        grid=(GRID,),
        in_specs=[col, col, col, col, full((31, 16)), wspec, bspec,
                  wspec, bspec, wspec, bspec, wspec, bspec, wspec, bspec],
        out_specs=[out64, out64, out32, out32, out64, out64],
        out_shape=[jax.ShapeDtypeStruct((N, H), jnp.float32),
                   jax.ShapeDtypeStruct((N, H), jnp.float32),
                   jax.ShapeDtypeStruct((N, H // 2), jnp.float32),
                   jax.ShapeDtypeStruct((N, H // 2), jnp.float32),
                   jax.ShapeDtypeStruct((N, H), jnp.float32),
                   jax.ShapeDtypeStruct((N, H), jnp.float32)],
    )(gtf, arf, dif, gnf, emb, W_in, b_in, Wq, bq, Wk, bk, Wv, bv, Ws, bs)


def _tcb_body(agga, aggb, sp, so, res, lg, lb, Wq, bq, Wk, bk, Wv, bv,
              Ws, bs, x1_o, q_o, k_o, va_o, vb_o, so_o):
    spv = sp[...]
    ssum = spv[:, 0:1] + spv[:, 1:2]
    inv = jnp.where(ssum > 0.0, 1.0 / ssum, 0.0)
    h = jnp.concatenate([agga[...], aggb[...]], axis=1) * inv + so[...]
    mu = jnp.mean(h, axis=1, keepdims=True)
    var = jnp.mean((h - mu) ** 2, axis=1, keepdims=True)
    h = (h - mu) * lax.rsqrt(var + 1e-5) * lg[...] + lb[...]
    x1 = h * jax.nn.sigmoid(h) + res[...]
    x1_o[...] = x1

    def lin(W, b):
        return jnp.dot(x1, W[...], preferred_element_type=jnp.float32) + b[...]

    q_o[...] = lin(Wq, bq)
    k_o[...] = lin(Wk, bk)
    v = lin(Wv, bv)
    va_o[...] = v[:, :H // 2]
    vb_o[...] = v[:, H // 2:]
    so_o[...] = lin(Ws, bs)


def _tcb(agga, aggb, sp, so, res, lg, lb, Wq, bq, Wk, bk, Wv, bv, Ws, bs):
    blk64 = pl.BlockSpec((BN, H), lambda i: (i, 0))
    blk32 = pl.BlockSpec((BN, H // 2), lambda i: (i, 0))
    blk2 = pl.BlockSpec((BN, 2), lambda i: (i, 0))
    full = lambda sh: pl.BlockSpec(sh, lambda i: tuple(0 for _ in sh))
    wspec = full((H, H))
    bspec = full((1, H))
    return pl.pallas_call(
        _tcb_body,
        grid=(GRID,),
        in_specs=[blk32, blk32, blk2, blk64, blk64, bspec, bspec,
                  wspec, bspec, wspec, bspec, wspec, bspec, wspec, bspec],
        out_specs=[blk64, blk64, blk64, blk32, blk32, blk64],
        out_shape=[jax.ShapeDtypeStruct((N, H), jnp.float32)] * 3
        + [jax.ShapeDtypeStruct((N, H // 2), jnp.float32)] * 2
        + [jax.ShapeDtypeStruct((N, H), jnp.float32)],
    )(agga, aggb, sp, so, res, lg, lb, Wq, bq, Wk, bk, Wv, bv, Ws, bs)


def _tcc_body(agga, aggb, sp, so, res, lg, lb, bf, bb, pb, W_bb, b_bb,
              W_th, b_th, W_rt, b_rt, W_a1, b_a1, W_a2, b_a2,
              th_o, rt_o, aux_o, sum_r, sq_r, mx_r, cnt_r):
    pid = pl.program_id(0)
    spv = sp[...]
    ssum = spv[:, 0:1] + spv[:, 1:2]
    inv = jnp.where(ssum > 0.0, 1.0 / ssum, 0.0)
    h = jnp.concatenate([agga[...], aggb[...]], axis=1) * inv + so[...]
    mu = jnp.mean(h, axis=1, keepdims=True)
    var = jnp.mean((h - mu) ** 2, axis=1, keepdims=True)
    h = (h - mu) * lax.rsqrt(var + 1e-5) * lg[...] + lb[...]
    x2 = h * jax.nn.sigmoid(h) + res[...]

    @pl.when(pid == 0)
    def _():
        sum_r[...] = jnp.zeros_like(sum_r)
        sq_r[...] = jnp.zeros_like(sq_r)
        mx_r[...] = jnp.full_like(mx_r, NEG)
        cnt_r[...] = jnp.zeros_like(cnt_r)

    b = bf[...]
    iot = lax.broadcasted_iota(jnp.int32, (1, NG), 1).astype(jnp.float32)
    oh = (b == iot).astype(jnp.float32)
    dn = (((0,), (0,)), ((), ()))
    sum_r[...] += lax.dot_general(oh, x2, dn,
                                  preferred_element_type=jnp.float32)
    sq_r[...] += lax.dot_general(oh, x2 * x2, dn,
                                 preferred_element_type=jnp.float32)
    cnt_r[...] += jnp.sum(oh, axis=0, keepdims=True)

    def upd(g, carry):
        ohg = (b == g.astype(jnp.float32))
        m = jnp.max(jnp.where(ohg, x2, NEG), axis=0, keepdims=True)
        mx_r[pl.ds(g, 1), :] = jnp.maximum(mx_r[pl.ds(g, 1), :], m)
        return carry
    lax.fori_loop(0, NG, upd, 0)

    @pl.when(pid == GRID - 1)
    def _():
        cnt = jnp.transpose(cnt_r[...])
        cnt1 = jnp.maximum(cnt, 1.0)
        mean = sum_r[...] / cnt1
        msq = sq_r[...] / cnt1
        std = jnp.sqrt(jnp.clip(msq - mean * mean, 1e-6, None))
        mx = jnp.where(cnt > 0.0, mx_r[...], 0.0)
        gnn = jnp.concatenate([mean, mx, std], axis=1)
        comb = jnp.concatenate([gnn, bb[...], pb[...]], axis=1)
        f = jnp.dot(comb, W_bb[...], preferred_element_type=jnp.float32)
        f = f + b_bb[...]
        f = f * jax.nn.sigmoid(f)
        th_o[...] = jnp.dot(f, W_th[...],
                            preferred_element_type=jnp.float32) + b_th[...]
        rt_o[...] = jnp.dot(f, W_rt[...],
                            preferred_element_type=jnp.float32) + b_rt[...]
        a1 = jnp.dot(gnn, W_a1[...],
                     preferred_element_type=jnp.float32) + b_a1[...]
        a1 = a1 * jax.nn.sigmoid(a1)
        aux_o[...] = jnp.dot(a1, W_a2[...],
                             preferred_element_type=jnp.float32) + b_a2[...]


def _tcc(agga, aggb, sp, so, res, lg, lb, bf, bb, pb, W_bb, b_bb,
         W_th, b_th, W_rt, b_rt, W_a1, b_a1, W_a2, b_a2):
    blk64 = pl.BlockSpec((BN, H), lambda i: (i, 0))
    blk32 = pl.BlockSpec((BN, H // 2), lambda i: (i, 0))
    blk2 = pl.BlockSpec((BN, 2), lambda i: (i, 0))
    col = pl.BlockSpec((BN, 1), lambda i: (i, 0))
    full = lambda sh: pl.BlockSpec(sh, lambda i: tuple(0 for _ in sh))
    return pl.pallas_call(
        _tcc_body,
        grid=(GRID,),
        in_specs=[blk32, blk32, blk2, blk64, blk64, full((1, H)),
                  full((1, H)), col, full((NG, 1)), full((NG, 1)),
                  full((3 * H + 2, H)), full((1, H)),
                  full((H, 10)), full((1, 10)),
                  full((H, 1)), full((1, 1)),
                  full((3 * H, H)), full((1, H)),
                  full((H, 32)), full((1, 32))],
        out_specs=[full((NG, 10)), full((NG, 1)), full((NG, 32))],
        out_shape=[jax.ShapeDtypeStruct((NG, 10), jnp.float32),
                   jax.ShapeDtypeStruct((NG, 1), jnp.float32),
                   jax.ShapeDtypeStruct((NG, 32), jnp.float32)],
        scratch_shapes=[pltpu.VMEM((NG, H), jnp.float32),
                        pltpu.VMEM((NG, H), jnp.float32),
                        pltpu.VMEM((NG, H), jnp.float32),
                        pltpu.VMEM((1, NG), jnp.float32)],
    )(agga, aggb, sp, so, res, lg, lb, bf, bb, pb, W_bb, b_bb,
      W_th, b_th, W_rt, b_rt, W_a1, b_a1, W_a2, b_a2)


def kernel(gate_type_idx, gate_arity, is_directional, gate_index_norm,
           edge_index, batch, backend_bit, precision_bit, emb, W_in, b_in,
           Wq0, bq0, Wk0, bk0, Wv0, bv0, Ws0, bs0, ln0_g, ln0_b,
           Wq1, bq1, Wk1, bk1, Wv1, bv1, Ws1, bs1, ln1_g, ln1_b,
           W_bb, b_bb, W_th, b_th, W_rt, b_rt, W_a1, b_a1, W_a2, b_a2):
    f32 = jnp.float32
    gtf = gate_type_idx.astype(f32).reshape(N, 1)
    arf = gate_arity.astype(f32).reshape(N, 1)
    dif = is_directional.astype(f32).reshape(N, 1)
    gnf = gate_index_norm.reshape(N, 1)
    bf = batch.astype(f32).reshape(N, 1)
    bb = backend_bit.reshape(NG, 1)
    pb = precision_bit.reshape(NG, 1)
    r1 = lambda v: v.reshape(1, -1)

    zpad = jnp.zeros((EP - E,), jnp.int32)
    src = jnp.concatenate([edge_index[0].astype(jnp.int32), zpad])
    dst = jnp.concatenate([edge_index[1].astype(jnp.int32), zpad])

    q0, k0, va0, vb0, so0, xproj = _tca(
        gtf, arf, dif, gnf, emb, W_in, r1(b_in), Wq0, r1(bq0), Wk0, r1(bk0),
        Wv0, r1(bv0), Ws0, r1(bs0))

    def edge_phase(q, k, va, vb):
        alpha, cpart = _sc1(q, k, src, dst)
        e, spart = _sc2(alpha, dst, cpart)
        vstk = jnp.concatenate([va, vb], axis=0)
        agg = _sc3(e, src, dst, vstk)
        sp = jnp.transpose(spart)[:N]
        return agg[0, :N], agg[1, :N], sp

    agga0, aggb0, sp0 = edge_phase(q0, k0, va0, vb0)

    x1, q1, k1, va1, vb1, so1 = _tcb(
        agga0, aggb0, sp0, so0, xproj, r1(ln0_g), r1(ln0_b),
        Wq1, r1(bq1), Wk1, r1(bk1), Wv1, r1(bv1), Ws1, r1(bs1))

    agga1, aggb1, sp1 = edge_phase(q1, k1, va1, vb1)

    th, rt, aux = _tcc(
        agga1, aggb1, sp1, so1, x1, r1(ln1_g), r1(ln1_b), bf, bb, pb,
        W_bb, r1(b_bb), W_th, r1(b_th), W_rt, r1(b_rt),
        W_a1, r1(b_a1), W_a2, r1(b_a2))
    return (th, rt[:, 0], aux)
